# Initial kernel scaffold; baseline (speedup 1.0000x reference)
import jax
import jax.numpy as jnp
from jax.experimental import pallas as pl


def kernel(input_xyz, coord_xyz, input_f, coord_f, Wf, bf, Ws, bs, gamma, beta, Wh, bh, Wo, bo):
    raise NotImplementedError("write your pallas kernel here")


# speedup vs baseline: 6.4916x; 6.4916x over previous
import functools

import jax
import jax.numpy as jnp
from jax import lax
from jax.experimental import pallas as pl
from jax.experimental.pallas import tpu as pltpu
from jax.experimental.pallas import tpu_sc as plsc

B, NP, DIM = 4, 1024, 128
K = 16
HID, NPROP = 64, 5
NG = 2
NNODE = B * NP
NEDGE = NG * NNODE * K

_INTERPRET = False


def _knn_body(xyz_ref, xyzt_ref, out_ref):
    d2 = jnp.zeros((NP, NP), jnp.float32)
    for c in range(3):
        col = xyz_ref[0, 0, :, c : c + 1]
        row = xyzt_ref[0, 0, c : c + 1, :]
        diff = col - row
        d2 = d2 + diff * diff
    ii = lax.broadcasted_iota(jnp.int32, (NP, NP), 0)
    jj = lax.broadcasted_iota(jnp.int32, (NP, NP), 1)
    d2 = jnp.where(ii == jj, d2 + 1e10, d2)
    cols = []
    for _ in range(K):
        m = jnp.min(d2, axis=1, keepdims=True)
        cand = jnp.where(d2 == m, jj, NP)
        sel = jnp.min(cand, axis=1, keepdims=True)
        cols.append(sel)
        d2 = jnp.where(jj == sel, jnp.float32(1e30), d2)
    g = pl.program_id(0)
    b = pl.program_id(1)
    base = (g * B + b) * NP
    out_ref[0, 0] = jnp.concatenate(cols, axis=1) + base


def _knn_call(xyzp, xyzt):
    return pl.pallas_call(
        _knn_body,
        grid=(NG, B),
        in_specs=[
            pl.BlockSpec((1, 1, NP, 8), lambda g, b: (g, b, 0, 0)),
            pl.BlockSpec((1, 1, 8, NP), lambda g, b: (g, b, 0, 0)),
        ],
        out_specs=pl.BlockSpec((1, 1, NP, K), lambda g, b: (g, b, 0, 0)),
        out_shape=jax.ShapeDtypeStruct((NG, B, NP, K), jnp.int32),
        interpret=_INTERPRET,
    )(xyzp, xyzt)


_NC, _NS = 2, 16
_NW = _NC * _NS
_CHUNK = 128


def _make_gather(n_rows, d):
    per_w = n_rows // _NW
    n_chunks = per_w // _CHUNK
    mesh = plsc.VectorSubcoreMesh(core_axis_name="c", subcore_axis_name="s")

    @functools.partial(
        pl.kernel,
        mesh=mesh,
        out_type=jax.ShapeDtypeStruct((n_rows, d), jnp.float32),
        scratch_types=[
            pltpu.VMEM((_CHUNK,), jnp.int32),
            pltpu.VMEM((_CHUNK, d), jnp.float32),
            pltpu.SemaphoreType.DMA,
        ],
    )
    def gk(idx_hbm, table_hbm, out_hbm, idx_v, rows_v, sem):
        wid = lax.axis_index("s") * _NC + lax.axis_index("c")
        base = wid * per_w

        def body(c, carry):
            off = base + c * _CHUNK
            pltpu.sync_copy(idx_hbm.at[pl.ds(off, _CHUNK)], idx_v)
            pltpu.async_copy(table_hbm.at[idx_v], rows_v, sem).wait()
            pltpu.sync_copy(rows_v, out_hbm.at[pl.ds(off, _CHUNK)])
            return carry

        lax.fori_loop(0, n_chunks, body, 0)

    return gk


_gather_kernel = None


def _gather_call(src, table):
    global _gather_kernel
    if _gather_kernel is None:
        _gather_kernel = _make_gather(NEDGE, DIM)
    return _gather_kernel(src, table)


_TN = 256
_NT = NG * NNODE // (_TN * NG)


def _msg_body(xd_ref, xg_ref, wft_ref, wfb_ref, wst_ref, wsb_ref,
              bf_ref, bs_ref, out_ref):
    xd = xd_ref[0, 0]
    xg = xg_ref[0, 0]
    p = jnp.dot(xd, wft_ref[...], preferred_element_type=jnp.float32) + bf_ref[...]
    r = jnp.dot(xd, wst_ref[...], preferred_element_type=jnp.float32) + bs_ref[...]
    q = jnp.dot(xg, wfb_ref[...], preferred_element_type=jnp.float32)
    s = jnp.dot(xg, wsb_ref[...], preferred_element_type=jnp.float32)
    a = q.reshape(_TN, K, DIM) + p.reshape(_TN, 1, DIM)
    t = s.reshape(_TN, K, DIM) + r.reshape(_TN, 1, DIM)
    sig = 1.0 / (1.0 + jnp.exp(-a))
    sp = jnp.maximum(t, 0.0) + jnp.log1p(jnp.exp(-jnp.abs(t)))
    out_ref[0, 0] = jnp.max(sig * sp, axis=1)


def _msg_call(xd, xg, wft, wfb, wst, wsb, bf, bs):
    nt = NNODE // _TN
    return pl.pallas_call(
        _msg_body,
        grid=(NG, nt),
        in_specs=[
            pl.BlockSpec((1, 1, _TN, DIM), lambda g, t: (g, t, 0, 0)),
            pl.BlockSpec((1, 1, _TN * K, DIM), lambda g, t: (g, t, 0, 0)),
            pl.BlockSpec((DIM, DIM), lambda g, t: (0, 0)),
            pl.BlockSpec((DIM, DIM), lambda g, t: (0, 0)),
            pl.BlockSpec((DIM, DIM), lambda g, t: (0, 0)),
            pl.BlockSpec((DIM, DIM), lambda g, t: (0, 0)),
            pl.BlockSpec((1, DIM), lambda g, t: (0, 0)),
            pl.BlockSpec((1, DIM), lambda g, t: (0, 0)),
        ],
        out_specs=pl.BlockSpec((1, 1, _TN, DIM), lambda g, t: (g, t, 0, 0)),
        out_shape=jax.ShapeDtypeStruct((NG, nt, _TN, DIM), jnp.float32),
        interpret=_INTERPRET,
    )(xd, xg, wft, wfb, wst, wsb, bf, bs)


def _norm_body(agg_ref, x_ref, gamma_ref, beta_ref, out_ref):
    a = agg_ref[0]
    mu = jnp.mean(a, axis=0, keepdims=True)
    var = jnp.mean((a - mu) ** 2, axis=0, keepdims=True)
    nrm = (a - mu) / jnp.sqrt(var + 1e-5) * gamma_ref[...] + beta_ref[...]
    out_ref[0] = x_ref[0] + nrm


def _norm_call(agg, x, gamma, beta):
    return pl.pallas_call(
        _norm_body,
        grid=(NG,),
        in_specs=[
            pl.BlockSpec((1, NNODE, DIM), lambda g: (g, 0, 0)),
            pl.BlockSpec((1, NNODE, DIM), lambda g: (g, 0, 0)),
            pl.BlockSpec((1, DIM), lambda g: (0, 0)),
            pl.BlockSpec((1, DIM), lambda g: (0, 0)),
        ],
        out_specs=pl.BlockSpec((1, NNODE, DIM), lambda g: (g, 0, 0)),
        out_shape=jax.ShapeDtypeStruct((NG, NNODE, DIM), jnp.float32),
        interpret=_INTERPRET,
    )(agg, x, gamma, beta)


def _cross_body(f0_ref, f1_ref, wh_ref, bh_ref, wo_ref, bo_ref,
                o0_ref, o1_ref):
    f0 = f0_ref[0]
    f1 = f1_ref[0]
    for l in range(NPROP):
        wh_t = wh_ref[l, :DIM, :]
        wh_b = wh_ref[l, DIM:, :]
        bh = bh_ref[l]
        wo = wo_ref[l]
        bo = bo_ref[l]
        s = lax.dot_general(f0, f1, (((1,), (1,)), ((), ())),
                            preferred_element_type=jnp.float32)
        m0 = jnp.max(s, axis=1, keepdims=True)
        e0 = jnp.exp(s - m0)
        a0 = e0 / jnp.sum(e0, axis=1, keepdims=True)
        m1 = jnp.max(s, axis=0, keepdims=True)
        e1 = jnp.exp(s - m1)
        a1 = e1 / jnp.sum(e1, axis=0, keepdims=True)
        att0 = jnp.dot(a0, f1, preferred_element_type=jnp.float32)
        att1 = lax.dot_general(a1, f0, (((0,), (0,)), ((), ())),
                               preferred_element_type=jnp.float32)
        mu0 = f0 - att0
        mu1 = f1 - att1
        h0 = jax.nn.relu(
            jnp.dot(f0, wh_t, preferred_element_type=jnp.float32)
            + jnp.dot(mu0, wh_b, preferred_element_type=jnp.float32) + bh)
        h1 = jax.nn.relu(
            jnp.dot(f1, wh_t, preferred_element_type=jnp.float32)
            + jnp.dot(mu1, wh_b, preferred_element_type=jnp.float32) + bh)
        f0 = f0 + jnp.dot(h0, wo, preferred_element_type=jnp.float32) + bo
        f1 = f1 + jnp.dot(h1, wo, preferred_element_type=jnp.float32) + bo
    o0_ref[0] = f0
    o1_ref[0] = f1


def _cross_call(f0, f1, wh, bh, wo, bo):
    return pl.pallas_call(
        _cross_body,
        grid=(B,),
        in_specs=[
            pl.BlockSpec((1, NP, DIM), lambda b: (b, 0, 0)),
            pl.BlockSpec((1, NP, DIM), lambda b: (b, 0, 0)),
            pl.BlockSpec((NPROP, 2 * DIM, HID), lambda b: (0, 0, 0)),
            pl.BlockSpec((NPROP, 1, HID), lambda b: (0, 0, 0)),
            pl.BlockSpec((NPROP, HID, DIM), lambda b: (0, 0, 0)),
            pl.BlockSpec((NPROP, 1, DIM), lambda b: (0, 0, 0)),
        ],
        out_specs=[
            pl.BlockSpec((1, NP, DIM), lambda b: (b, 0, 0)),
            pl.BlockSpec((1, NP, DIM), lambda b: (b, 0, 0)),
        ],
        out_shape=[
            jax.ShapeDtypeStruct((B, NP, DIM), jnp.float32),
            jax.ShapeDtypeStruct((B, NP, DIM), jnp.float32),
        ],
        interpret=_INTERPRET,
    )(f0, f1, wh, bh, wo, bo)


def kernel(input_xyz, coord_xyz, input_f, coord_f, Wf, bf, Ws, bs,
           gamma, beta, Wh, bh, Wo, bo):
    xyz = jnp.stack([input_xyz, coord_xyz])
    pad = jnp.zeros((NG, B, NP, 5), jnp.float32)
    xyzp = jnp.concatenate([xyz, pad], axis=-1)
    xyzt = jnp.swapaxes(xyzp, 2, 3)

    x = jnp.stack([input_f.reshape(NNODE, DIM),
                   coord_f.reshape(NNODE, DIM)])

    idx = _knn_call(xyzp, xyzt)
    src = idx.reshape(-1)
    table = x.reshape(NG * NNODE, DIM)
    xg = _gather_call(src, table)

    nt = NNODE // _TN
    agg = _msg_call(
        x.reshape(NG, nt, _TN, DIM),
        xg.reshape(NG, nt, _TN * K, DIM),
        Wf[:DIM], Wf[DIM:], Ws[:DIM], Ws[DIM:],
        bf.reshape(1, DIM), bs.reshape(1, DIM),
    )

    f = _norm_call(agg.reshape(NG, NNODE, DIM), x,
                   gamma.reshape(1, DIM), beta.reshape(1, DIM))

    fb = f.reshape(NG, B, NP, DIM)
    o0, o1 = _cross_call(fb[0], fb[1], Wh, bh.reshape(NPROP, 1, HID),
                         Wo, bo.reshape(NPROP, 1, DIM))
    return o0.reshape(-1, DIM), o1.reshape(-1, DIM)


# speedup vs baseline: 6.7660x; 1.0423x over previous
import functools

import jax
import jax.numpy as jnp
from jax import lax
from jax.experimental import pallas as pl
from jax.experimental.pallas import tpu as pltpu
from jax.experimental.pallas import tpu_sc as plsc

B, NP, DIM = 4, 1024, 128
K = 16
HID, NPROP = 64, 5
NG = 2
NNODE = B * NP
NEDGE = NG * NNODE * K

_INTERPRET = False


def _knn_body(xyz_ref, xyzt_ref, out_ref):
    d2 = jnp.zeros((NP, NP), jnp.float32)
    for c in range(3):
        col = xyz_ref[0, 0, :, c : c + 1]
        row = xyzt_ref[0, 0, c : c + 1, :]
        diff = col - row
        d2 = d2 + diff * diff
    ii = lax.broadcasted_iota(jnp.int32, (NP, NP), 0)
    jj = lax.broadcasted_iota(jnp.int32, (NP, NP), 1)
    d2 = jnp.where(ii == jj, d2 + 1e10, d2)
    rows = []
    for _ in range(K):
        m = jnp.min(d2, axis=0, keepdims=True)
        cand = jnp.where(d2 == m, ii, NP)
        sel = jnp.min(cand, axis=0, keepdims=True)
        rows.append(sel)
        d2 = jnp.where(ii == sel, jnp.float32(1e30), d2)
    g = pl.program_id(0)
    b = pl.program_id(1)
    base = (g * B + b) * NP
    out_ref[0, 0] = jnp.concatenate(rows, axis=0) + base


def _knn_call(xyzp, xyzt):
    return pl.pallas_call(
        _knn_body,
        grid=(NG, B),
        in_specs=[
            pl.BlockSpec((1, 1, NP, 8), lambda g, b: (g, b, 0, 0)),
            pl.BlockSpec((1, 1, 8, NP), lambda g, b: (g, b, 0, 0)),
        ],
        out_specs=pl.BlockSpec((1, 1, K, NP), lambda g, b: (g, b, 0, 0)),
        out_shape=jax.ShapeDtypeStruct((NG, B, K, NP), jnp.int32),
        interpret=_INTERPRET,
    )(xyzp, xyzt)


_NC, _NS = 2, 16
_NW = _NC * _NS
_CHUNK = 128


def _make_gather(n_rows, d):
    per_w = n_rows // _NW
    n_chunks = per_w // _CHUNK
    mesh = plsc.VectorSubcoreMesh(core_axis_name="c", subcore_axis_name="s")

    @functools.partial(
        pl.kernel,
        mesh=mesh,
        out_type=jax.ShapeDtypeStruct((n_rows, d), jnp.float32),
        scratch_types=[
            pltpu.VMEM((_CHUNK,), jnp.int32),
            pltpu.VMEM((_CHUNK, d), jnp.float32),
            pltpu.SemaphoreType.DMA,
        ],
    )
    def gk(idx_hbm, table_hbm, out_hbm, idx_v, rows_v, sem):
        wid = lax.axis_index("s") * _NC + lax.axis_index("c")
        base = wid * per_w

        def body(c, carry):
            off = base + c * _CHUNK
            pltpu.sync_copy(idx_hbm.at[pl.ds(off, _CHUNK)], idx_v)
            pltpu.async_copy(table_hbm.at[idx_v], rows_v, sem).wait()
            pltpu.sync_copy(rows_v, out_hbm.at[pl.ds(off, _CHUNK)])
            return carry

        lax.fori_loop(0, n_chunks, body, 0)

    return gk


_gather_kernel = None


def _gather_call(src, table):
    global _gather_kernel
    if _gather_kernel is None:
        _gather_kernel = _make_gather(NEDGE, DIM)
    return _gather_kernel(src, table)


_TN = 256
_NT = NG * NNODE // (_TN * NG)


def _msg_body(xd_ref, xg_ref, wft_ref, wfb_ref, wst_ref, wsb_ref,
              bf_ref, bs_ref, out_ref):
    xd = xd_ref[0, 0]
    xg = xg_ref[0, 0, :, :, :].reshape(K * _TN, DIM)
    p = jnp.dot(xd, wft_ref[...], preferred_element_type=jnp.float32) + bf_ref[...]
    r = jnp.dot(xd, wst_ref[...], preferred_element_type=jnp.float32) + bs_ref[...]
    q = jnp.dot(xg, wfb_ref[...], preferred_element_type=jnp.float32)
    s = jnp.dot(xg, wsb_ref[...], preferred_element_type=jnp.float32)
    a = q.reshape(K, _TN, DIM) + p.reshape(1, _TN, DIM)
    t = s.reshape(K, _TN, DIM) + r.reshape(1, _TN, DIM)
    sig = 1.0 / (1.0 + jnp.exp(-a))
    sp = jnp.maximum(t, 0.0) + jnp.log1p(jnp.exp(-jnp.abs(t)))
    out_ref[0, 0] = jnp.max(sig * sp, axis=0)


def _msg_call(xd, xg, wft, wfb, wst, wsb, bf, bs):
    nt = NNODE // _TN
    npb = NP // _TN
    return pl.pallas_call(
        _msg_body,
        grid=(NG, nt),
        in_specs=[
            pl.BlockSpec((1, 1, _TN, DIM), lambda g, t: (g, t, 0, 0)),
            pl.BlockSpec((1, 1, K, _TN, DIM),
                         lambda g, t: (g, t // npb, 0, t % npb, 0)),
            pl.BlockSpec((DIM, DIM), lambda g, t: (0, 0)),
            pl.BlockSpec((DIM, DIM), lambda g, t: (0, 0)),
            pl.BlockSpec((DIM, DIM), lambda g, t: (0, 0)),
            pl.BlockSpec((DIM, DIM), lambda g, t: (0, 0)),
            pl.BlockSpec((1, DIM), lambda g, t: (0, 0)),
            pl.BlockSpec((1, DIM), lambda g, t: (0, 0)),
        ],
        out_specs=pl.BlockSpec((1, 1, _TN, DIM), lambda g, t: (g, t, 0, 0)),
        out_shape=jax.ShapeDtypeStruct((NG, nt, _TN, DIM), jnp.float32),
        interpret=_INTERPRET,
    )(xd, xg, wft, wfb, wst, wsb, bf, bs)


def _norm_body(agg_ref, x_ref, gamma_ref, beta_ref, out_ref):
    a = agg_ref[0]
    mu = jnp.mean(a, axis=0, keepdims=True)
    var = jnp.mean((a - mu) ** 2, axis=0, keepdims=True)
    nrm = (a - mu) / jnp.sqrt(var + 1e-5) * gamma_ref[...] + beta_ref[...]
    out_ref[0] = x_ref[0] + nrm


def _norm_call(agg, x, gamma, beta):
    return pl.pallas_call(
        _norm_body,
        grid=(NG,),
        in_specs=[
            pl.BlockSpec((1, NNODE, DIM), lambda g: (g, 0, 0)),
            pl.BlockSpec((1, NNODE, DIM), lambda g: (g, 0, 0)),
            pl.BlockSpec((1, DIM), lambda g: (0, 0)),
            pl.BlockSpec((1, DIM), lambda g: (0, 0)),
        ],
        out_specs=pl.BlockSpec((1, NNODE, DIM), lambda g: (g, 0, 0)),
        out_shape=jax.ShapeDtypeStruct((NG, NNODE, DIM), jnp.float32),
        interpret=_INTERPRET,
    )(agg, x, gamma, beta)


def _cross_body(f0_ref, f1_ref, wh_ref, bh_ref, wo_ref, bo_ref,
                o0_ref, o1_ref):
    f0 = f0_ref[0]
    f1 = f1_ref[0]
    for l in range(NPROP):
        wh_t = wh_ref[l, :DIM, :]
        wh_b = wh_ref[l, DIM:, :]
        bh = bh_ref[l]
        wo = wo_ref[l]
        bo = bo_ref[l]
        s = lax.dot_general(f0, f1, (((1,), (1,)), ((), ())),
                            preferred_element_type=jnp.float32)
        m0 = jnp.max(s, axis=1, keepdims=True)
        e0 = jnp.exp(s - m0)
        a0 = e0 / jnp.sum(e0, axis=1, keepdims=True)
        m1 = jnp.max(s, axis=0, keepdims=True)
        e1 = jnp.exp(s - m1)
        a1 = e1 / jnp.sum(e1, axis=0, keepdims=True)
        att0 = jnp.dot(a0, f1, preferred_element_type=jnp.float32)
        att1 = lax.dot_general(a1, f0, (((0,), (0,)), ((), ())),
                               preferred_element_type=jnp.float32)
        mu0 = f0 - att0
        mu1 = f1 - att1
        h0 = jax.nn.relu(
            jnp.dot(f0, wh_t, preferred_element_type=jnp.float32)
            + jnp.dot(mu0, wh_b, preferred_element_type=jnp.float32) + bh)
        h1 = jax.nn.relu(
            jnp.dot(f1, wh_t, preferred_element_type=jnp.float32)
            + jnp.dot(mu1, wh_b, preferred_element_type=jnp.float32) + bh)
        f0 = f0 + jnp.dot(h0, wo, preferred_element_type=jnp.float32) + bo
        f1 = f1 + jnp.dot(h1, wo, preferred_element_type=jnp.float32) + bo
    o0_ref[0] = f0
    o1_ref[0] = f1


def _cross_call(f0, f1, wh, bh, wo, bo):
    return pl.pallas_call(
        _cross_body,
        grid=(B,),
        in_specs=[
            pl.BlockSpec((1, NP, DIM), lambda b: (b, 0, 0)),
            pl.BlockSpec((1, NP, DIM), lambda b: (b, 0, 0)),
            pl.BlockSpec((NPROP, 2 * DIM, HID), lambda b: (0, 0, 0)),
            pl.BlockSpec((NPROP, 1, HID), lambda b: (0, 0, 0)),
            pl.BlockSpec((NPROP, HID, DIM), lambda b: (0, 0, 0)),
            pl.BlockSpec((NPROP, 1, DIM), lambda b: (0, 0, 0)),
        ],
        out_specs=[
            pl.BlockSpec((1, NP, DIM), lambda b: (b, 0, 0)),
            pl.BlockSpec((1, NP, DIM), lambda b: (b, 0, 0)),
        ],
        out_shape=[
            jax.ShapeDtypeStruct((B, NP, DIM), jnp.float32),
            jax.ShapeDtypeStruct((B, NP, DIM), jnp.float32),
        ],
        interpret=_INTERPRET,
    )(f0, f1, wh, bh, wo, bo)


def kernel(input_xyz, coord_xyz, input_f, coord_f, Wf, bf, Ws, bs,
           gamma, beta, Wh, bh, Wo, bo):
    xyz = jnp.stack([input_xyz, coord_xyz])
    pad = jnp.zeros((NG, B, NP, 5), jnp.float32)
    xyzp = jnp.concatenate([xyz, pad], axis=-1)
    xyzt = jnp.swapaxes(xyzp, 2, 3)

    x = jnp.stack([input_f.reshape(NNODE, DIM),
                   coord_f.reshape(NNODE, DIM)])

    idx = _knn_call(xyzp, xyzt)
    src = idx.reshape(-1)
    table = x.reshape(NG * NNODE, DIM)
    xg = _gather_call(src, table)

    nt = NNODE // _TN
    agg = _msg_call(
        x.reshape(NG, nt, _TN, DIM),
        xg.reshape(NG, B, K, NP, DIM),
        Wf[:DIM], Wf[DIM:], Ws[:DIM], Ws[DIM:],
        bf.reshape(1, DIM), bs.reshape(1, DIM),
    )

    f = _norm_call(agg.reshape(NG, NNODE, DIM), x,
                   gamma.reshape(1, DIM), beta.reshape(1, DIM))

    fb = f.reshape(NG, B, NP, DIM)
    o0, o1 = _cross_call(fb[0], fb[1], Wh, bh.reshape(NPROP, 1, HID),
                         Wo, bo.reshape(NPROP, 1, DIM))
    return o0.reshape(-1, DIM), o1.reshape(-1, DIM)


# speedup vs baseline: 7.7051x; 1.1388x over previous
import functools

import jax
import jax.numpy as jnp
from jax import lax
from jax.experimental import pallas as pl
from jax.experimental.pallas import tpu as pltpu
from jax.experimental.pallas import tpu_sc as plsc

B, NP, DIM = 4, 1024, 128
K = 16
HID, NPROP = 64, 5
NG = 2
NNODE = B * NP
NEDGE = NNODE * K

_INTERPRET = False


def _knn_body(xyz_ref, xyzt_ref, out_ref, *, gbase):
    d2 = jnp.zeros((NP, NP), jnp.float32)
    for c in range(3):
        col = xyz_ref[0, :, c : c + 1]
        row = xyzt_ref[0, c : c + 1, :]
        diff = col - row
        d2 = d2 + diff * diff
    ii = lax.broadcasted_iota(jnp.int32, (NP, NP), 0)
    jj = lax.broadcasted_iota(jnp.int32, (NP, NP), 1)
    d2 = jnp.where(ii == jj, d2 + 1e10, d2)
    rows = []
    m = jnp.min(d2, axis=0, keepdims=True)
    for r in range(K):
        cand = jnp.where(d2 == m, ii, NP)
        sel = jnp.min(cand, axis=0, keepdims=True)
        rows.append(sel)
        if r + 1 < K:
            d2 = jnp.where(ii == sel, jnp.float32(1e30), d2)
            m = jnp.min(d2, axis=0, keepdims=True)
    b = pl.program_id(0)
    base = gbase + b * NP
    out_ref[0] = jnp.concatenate(rows, axis=0) + base


def _knn_call(xyzp, xyzt, gbase):
    return pl.pallas_call(
        functools.partial(_knn_body, gbase=gbase),
        grid=(B,),
        in_specs=[
            pl.BlockSpec((1, NP, 8), lambda b: (b, 0, 0)),
            pl.BlockSpec((1, 8, NP), lambda b: (b, 0, 0)),
        ],
        out_specs=pl.BlockSpec((1, K, NP), lambda b: (b, 0, 0)),
        out_shape=jax.ShapeDtypeStruct((B, K, NP), jnp.int32),
        interpret=_INTERPRET,
    )(xyzp, xyzt)


_NC, _NS = 2, 16
_NW = _NC * _NS
_CHUNK = 128


def _make_gather(n_rows, d):
    per_w = n_rows // _NW
    n_chunks = per_w // _CHUNK
    mesh = plsc.VectorSubcoreMesh(core_axis_name="c", subcore_axis_name="s")

    @functools.partial(
        pl.kernel,
        mesh=mesh,
        out_type=jax.ShapeDtypeStruct((n_rows, d), jnp.float32),
        scratch_types=[
            pltpu.VMEM((_CHUNK,), jnp.int32),
            pltpu.VMEM((_CHUNK, d), jnp.float32),
            pltpu.SemaphoreType.DMA,
        ],
    )
    def gk(idx_hbm, table_hbm, out_hbm, idx_v, rows_v, sem):
        wid = lax.axis_index("s") * _NC + lax.axis_index("c")
        base = wid * per_w

        def body(c, carry):
            off = base + c * _CHUNK
            pltpu.sync_copy(idx_hbm.at[pl.ds(off, _CHUNK)], idx_v)
            pltpu.async_copy(table_hbm.at[idx_v], rows_v, sem).wait()
            pltpu.sync_copy(rows_v, out_hbm.at[pl.ds(off, _CHUNK)])
            return carry

        lax.fori_loop(0, n_chunks, body, 0)

    return gk


_gather_kernel = None


def _gather_call(src, table):
    global _gather_kernel
    if _gather_kernel is None:
        _gather_kernel = _make_gather(NEDGE, DIM)
    return _gather_kernel(src, table)


_TN = 256


def _msg_body(xd_ref, xg_ref, wft_ref, wfb_ref, wst_ref, wsb_ref,
              bf_ref, bs_ref, out_ref):
    xd = xd_ref[0]
    xg = xg_ref[0, :, :, :].reshape(K * _TN, DIM)
    p = jnp.dot(xd, wft_ref[...], preferred_element_type=jnp.float32) + bf_ref[...]
    r = jnp.dot(xd, wst_ref[...], preferred_element_type=jnp.float32) + bs_ref[...]
    q = jnp.dot(xg, wfb_ref[...], preferred_element_type=jnp.float32)
    s = jnp.dot(xg, wsb_ref[...], preferred_element_type=jnp.float32)
    a = q.reshape(K, _TN, DIM) + p.reshape(1, _TN, DIM)
    t = s.reshape(K, _TN, DIM) + r.reshape(1, _TN, DIM)
    sig = 1.0 / (1.0 + jnp.exp(-a))
    sp = jnp.maximum(t, 0.0) + jnp.log1p(jnp.exp(-jnp.abs(t)))
    out_ref[0] = jnp.max(sig * sp, axis=0)


def _msg_call(xd, xg, wft, wfb, wst, wsb, bf, bs):
    nt = NNODE // _TN
    npb = NP // _TN
    return pl.pallas_call(
        _msg_body,
        grid=(nt,),
        in_specs=[
            pl.BlockSpec((1, _TN, DIM), lambda t: (t, 0, 0)),
            pl.BlockSpec((1, K, _TN, DIM),
                         lambda t: (t // npb, 0, t % npb, 0)),
            pl.BlockSpec((DIM, DIM), lambda t: (0, 0)),
            pl.BlockSpec((DIM, DIM), lambda t: (0, 0)),
            pl.BlockSpec((DIM, DIM), lambda t: (0, 0)),
            pl.BlockSpec((DIM, DIM), lambda t: (0, 0)),
            pl.BlockSpec((1, DIM), lambda t: (0, 0)),
            pl.BlockSpec((1, DIM), lambda t: (0, 0)),
        ],
        out_specs=pl.BlockSpec((1, _TN, DIM), lambda t: (t, 0, 0)),
        out_shape=jax.ShapeDtypeStruct((nt, _TN, DIM), jnp.float32),
        interpret=_INTERPRET,
    )(xd, xg, wft, wfb, wst, wsb, bf, bs)


def _norm_body(agg_ref, x_ref, gamma_ref, beta_ref, out_ref):
    a = agg_ref[...]
    mu = jnp.mean(a, axis=0, keepdims=True)
    var = jnp.mean((a - mu) ** 2, axis=0, keepdims=True)
    nrm = (a - mu) / jnp.sqrt(var + 1e-5) * gamma_ref[...] + beta_ref[...]
    out_ref[...] = x_ref[...] + nrm


def _norm_call(agg, x, gamma, beta):
    return pl.pallas_call(
        _norm_body,
        in_specs=[
            pl.BlockSpec((NNODE, DIM), lambda: (0, 0)),
            pl.BlockSpec((NNODE, DIM), lambda: (0, 0)),
            pl.BlockSpec((1, DIM), lambda: (0, 0)),
            pl.BlockSpec((1, DIM), lambda: (0, 0)),
        ],
        out_specs=pl.BlockSpec((NNODE, DIM), lambda: (0, 0)),
        out_shape=jax.ShapeDtypeStruct((NNODE, DIM), jnp.float32),
        interpret=_INTERPRET,
    )(agg, x, gamma, beta)


def _cross_body(f0_ref, f1_ref, wh_ref, bh_ref, wo_ref, bo_ref,
                o0_ref, o1_ref):
    f0 = f0_ref[0]
    f1 = f1_ref[0]
    for l in range(NPROP):
        wh_t = wh_ref[l, :DIM, :]
        wh_b = wh_ref[l, DIM:, :]
        bh = bh_ref[l]
        wo = wo_ref[l]
        bo = bo_ref[l]
        s = lax.dot_general(f0, f1, (((1,), (1,)), ((), ())),
                            preferred_element_type=jnp.float32)
        m0 = jnp.max(s, axis=1, keepdims=True)
        e0 = jnp.exp(s - m0)
        a0 = e0 / jnp.sum(e0, axis=1, keepdims=True)
        m1 = jnp.max(s, axis=0, keepdims=True)
        e1 = jnp.exp(s - m1)
        a1 = e1 / jnp.sum(e1, axis=0, keepdims=True)
        att0 = jnp.dot(a0, f1, preferred_element_type=jnp.float32)
        att1 = lax.dot_general(a1, f0, (((0,), (0,)), ((), ())),
                               preferred_element_type=jnp.float32)
        mu0 = f0 - att0
        mu1 = f1 - att1
        h0 = jax.nn.relu(
            jnp.dot(f0, wh_t, preferred_element_type=jnp.float32)
            + jnp.dot(mu0, wh_b, preferred_element_type=jnp.float32) + bh)
        h1 = jax.nn.relu(
            jnp.dot(f1, wh_t, preferred_element_type=jnp.float32)
            + jnp.dot(mu1, wh_b, preferred_element_type=jnp.float32) + bh)
        f0 = f0 + jnp.dot(h0, wo, preferred_element_type=jnp.float32) + bo
        f1 = f1 + jnp.dot(h1, wo, preferred_element_type=jnp.float32) + bo
    o0_ref[0] = f0
    o1_ref[0] = f1


def _cross_call(f0, f1, wh, bh, wo, bo):
    return pl.pallas_call(
        _cross_body,
        grid=(B,),
        in_specs=[
            pl.BlockSpec((1, NP, DIM), lambda b: (b, 0, 0)),
            pl.BlockSpec((1, NP, DIM), lambda b: (b, 0, 0)),
            pl.BlockSpec((NPROP, 2 * DIM, HID), lambda b: (0, 0, 0)),
            pl.BlockSpec((NPROP, 1, HID), lambda b: (0, 0, 0)),
            pl.BlockSpec((NPROP, HID, DIM), lambda b: (0, 0, 0)),
            pl.BlockSpec((NPROP, 1, DIM), lambda b: (0, 0, 0)),
        ],
        out_specs=[
            pl.BlockSpec((1, NP, DIM), lambda b: (b, 0, 0)),
            pl.BlockSpec((1, NP, DIM), lambda b: (b, 0, 0)),
        ],
        out_shape=[
            jax.ShapeDtypeStruct((B, NP, DIM), jnp.float32),
            jax.ShapeDtypeStruct((B, NP, DIM), jnp.float32),
        ],
        interpret=_INTERPRET,
    )(f0, f1, wh, bh, wo, bo)


def kernel(input_xyz, coord_xyz, input_f, coord_f, Wf, bf, Ws, bs,
           gamma, beta, Wh, bh, Wo, bo):
    x0 = input_f.reshape(NNODE, DIM)
    x1 = coord_f.reshape(NNODE, DIM)
    table = jnp.concatenate([x0, x1], axis=0)

    wft, wfb = Wf[:DIM], Wf[DIM:]
    wst, wsb = Ws[:DIM], Ws[DIM:]
    bf2 = bf.reshape(1, DIM)
    bs2 = bs.reshape(1, DIM)
    nt = NNODE // _TN

    fs = []
    pad = jnp.zeros((B, NP, 5), jnp.float32)
    for g, (xyz, x) in enumerate(((input_xyz, x0), (coord_xyz, x1))):
        xyzp = jnp.concatenate([xyz, pad], axis=-1)
        xyzt = jnp.swapaxes(xyzp, 1, 2)
        idx = _knn_call(xyzp, xyzt, g * NNODE)
        xg = _gather_call(idx.reshape(-1), table)
        agg = _msg_call(
            x.reshape(nt, _TN, DIM),
            xg.reshape(B, K, NP, DIM),
            wft, wfb, wst, wsb, bf2, bs2,
        )
        fs.append(_norm_call(agg.reshape(NNODE, DIM), x,
                             gamma.reshape(1, DIM), beta.reshape(1, DIM)))

    o0, o1 = _cross_call(fs[0].reshape(B, NP, DIM), fs[1].reshape(B, NP, DIM),
                         Wh, bh.reshape(NPROP, 1, HID),
                         Wo, bo.reshape(NPROP, 1, DIM))
    return o0.reshape(-1, DIM), o1.reshape(-1, DIM)


# speedup vs baseline: 7.8425x; 1.0178x over previous
import functools

import jax
import jax.numpy as jnp
from jax import lax
from jax.experimental import pallas as pl
from jax.experimental.pallas import tpu as pltpu
from jax.experimental.pallas import tpu_sc as plsc

B, NP, DIM = 4, 1024, 128
K = 16
HID, NPROP = 64, 5
NG = 2
NNODE = B * NP
NEDGE = NNODE * K

_INTERPRET = False


def _knn_body(xyz_ref, xyzt_ref, out_ref, *, gbase):
    d2 = jnp.zeros((NP, NP), jnp.float32)
    for c in range(3):
        col = xyz_ref[0, :, c : c + 1]
        row = xyzt_ref[0, c : c + 1, :]
        diff = col - row
        d2 = d2 + diff * diff
    ii = lax.broadcasted_iota(jnp.int32, (NP, NP), 0)
    jj = lax.broadcasted_iota(jnp.int32, (NP, NP), 1)
    d2 = jnp.where(ii == jj, d2 + 1e10, d2)
    rows = []
    m = jnp.min(d2, axis=0, keepdims=True)
    for r in range(K):
        cand = jnp.where(d2 == m, ii, NP)
        sel = jnp.min(cand, axis=0, keepdims=True)
        rows.append(sel)
        if r + 1 < K:
            d2 = jnp.where(ii == sel, jnp.float32(1e30), d2)
            m = jnp.min(d2, axis=0, keepdims=True)
    b = pl.program_id(0)
    base = gbase + b * NP
    out_ref[0] = jnp.concatenate(rows, axis=0) + base


def _knn_call(xyzp, xyzt, gbase):
    return pl.pallas_call(
        functools.partial(_knn_body, gbase=gbase),
        grid=(B,),
        in_specs=[
            pl.BlockSpec((1, NP, 8), lambda b: (b, 0, 0)),
            pl.BlockSpec((1, 8, NP), lambda b: (b, 0, 0)),
        ],
        out_specs=pl.BlockSpec((1, K, NP), lambda b: (b, 0, 0)),
        out_shape=jax.ShapeDtypeStruct((B, K, NP), jnp.int32),
        interpret=_INTERPRET,
    )(xyzp, xyzt)


_NC, _NS = 2, 16
_NW = _NC * _NS
_CHUNK = 128


def _make_gather(n_rows, d):
    per_w = n_rows // _NW
    n_chunks = per_w // _CHUNK
    mesh = plsc.VectorSubcoreMesh(core_axis_name="c", subcore_axis_name="s")

    @functools.partial(
        pl.kernel,
        mesh=mesh,
        out_type=jax.ShapeDtypeStruct((n_rows, d), jnp.float32),
        scratch_types=[
            pltpu.VMEM((per_w,), jnp.int32),
            pltpu.VMEM((_CHUNK, d), jnp.float32),
            pltpu.VMEM((_CHUNK, d), jnp.float32),
            pltpu.SemaphoreType.DMA,
            pltpu.SemaphoreType.DMA,
            pltpu.SemaphoreType.DMA,
            pltpu.SemaphoreType.DMA,
        ],
    )
    def gk(idx_hbm, table_hbm, out_hbm, idx_v, b0, b1, sg0, sg1, ss0, ss1):
        wid = lax.axis_index("s") * _NC + lax.axis_index("c")
        base = wid * per_w
        bufs = (b0, b1)
        gsem = (sg0, sg1)
        ssem = (ss0, ss1)
        pltpu.sync_copy(idx_hbm.at[pl.ds(base, per_w)], idx_v)
        cp_g = {}
        cp_s = {}
        cp_g[0] = pltpu.async_copy(
            table_hbm.at[idx_v.at[pl.ds(0, _CHUNK)]], bufs[0], gsem[0])
        for c in range(n_chunks):
            cp_g[c].wait()
            cp_s[c] = pltpu.async_copy(
                bufs[c % 2], out_hbm.at[pl.ds(base + c * _CHUNK, _CHUNK)],
                ssem[c % 2])
            if c + 1 < n_chunks:
                if c >= 1:
                    cp_s[c - 1].wait()
                cp_g[c + 1] = pltpu.async_copy(
                    table_hbm.at[idx_v.at[pl.ds((c + 1) * _CHUNK, _CHUNK)]],
                    bufs[(c + 1) % 2], gsem[(c + 1) % 2])
        cp_s[n_chunks - 1].wait()
        if n_chunks >= 2:
            cp_s[n_chunks - 2].wait()

    return gk


_gather_kernel = None


def _gather_call(src, table):
    global _gather_kernel
    if _gather_kernel is None:
        _gather_kernel = _make_gather(NEDGE, DIM)
    return _gather_kernel(src, table)


_TN = 256


def _msg_body(xd_ref, xg_ref, wft_ref, wfb_ref, wst_ref, wsb_ref,
              bf_ref, bs_ref, out_ref):
    xd = xd_ref[0]
    xg = xg_ref[0, :, :, :].reshape(K * _TN, DIM)
    p = jnp.dot(xd, wft_ref[...], preferred_element_type=jnp.float32) + bf_ref[...]
    r = jnp.dot(xd, wst_ref[...], preferred_element_type=jnp.float32) + bs_ref[...]
    q = jnp.dot(xg, wfb_ref[...], preferred_element_type=jnp.float32)
    s = jnp.dot(xg, wsb_ref[...], preferred_element_type=jnp.float32)
    a = q.reshape(K, _TN, DIM) + p.reshape(1, _TN, DIM)
    t = s.reshape(K, _TN, DIM) + r.reshape(1, _TN, DIM)
    sig = 1.0 / (1.0 + jnp.exp(-a))
    sp = jnp.maximum(t, 0.0) + jnp.log1p(jnp.exp(-jnp.abs(t)))
    out_ref[0] = jnp.max(sig * sp, axis=0)


def _msg_call(xd, xg, wft, wfb, wst, wsb, bf, bs):
    nt = NNODE // _TN
    npb = NP // _TN
    return pl.pallas_call(
        _msg_body,
        grid=(nt,),
        in_specs=[
            pl.BlockSpec((1, _TN, DIM), lambda t: (t, 0, 0)),
            pl.BlockSpec((1, K, _TN, DIM),
                         lambda t: (t // npb, 0, t % npb, 0)),
            pl.BlockSpec((DIM, DIM), lambda t: (0, 0)),
            pl.BlockSpec((DIM, DIM), lambda t: (0, 0)),
            pl.BlockSpec((DIM, DIM), lambda t: (0, 0)),
            pl.BlockSpec((DIM, DIM), lambda t: (0, 0)),
            pl.BlockSpec((1, DIM), lambda t: (0, 0)),
            pl.BlockSpec((1, DIM), lambda t: (0, 0)),
        ],
        out_specs=pl.BlockSpec((1, _TN, DIM), lambda t: (t, 0, 0)),
        out_shape=jax.ShapeDtypeStruct((nt, _TN, DIM), jnp.float32),
        interpret=_INTERPRET,
    )(xd, xg, wft, wfb, wst, wsb, bf, bs)


def _norm_body(agg_ref, x_ref, gamma_ref, beta_ref, out_ref):
    a = agg_ref[...]
    mu = jnp.mean(a, axis=0, keepdims=True)
    var = jnp.mean((a - mu) ** 2, axis=0, keepdims=True)
    nrm = (a - mu) / jnp.sqrt(var + 1e-5) * gamma_ref[...] + beta_ref[...]
    out_ref[...] = x_ref[...] + nrm


def _norm_call(agg, x, gamma, beta):
    return pl.pallas_call(
        _norm_body,
        in_specs=[
            pl.BlockSpec((NNODE, DIM), lambda: (0, 0)),
            pl.BlockSpec((NNODE, DIM), lambda: (0, 0)),
            pl.BlockSpec((1, DIM), lambda: (0, 0)),
            pl.BlockSpec((1, DIM), lambda: (0, 0)),
        ],
        out_specs=pl.BlockSpec((NNODE, DIM), lambda: (0, 0)),
        out_shape=jax.ShapeDtypeStruct((NNODE, DIM), jnp.float32),
        interpret=_INTERPRET,
    )(agg, x, gamma, beta)


def _cross_body(f0_ref, f1_ref, wh_ref, bh_ref, wo_ref, bo_ref,
                o0_ref, o1_ref):
    f0 = f0_ref[0]
    f1 = f1_ref[0]
    for l in range(NPROP):
        wh_t = wh_ref[l, :DIM, :]
        wh_b = wh_ref[l, DIM:, :]
        bh = bh_ref[l]
        wo = wo_ref[l]
        bo = bo_ref[l]
        s = lax.dot_general(f0, f1, (((1,), (1,)), ((), ())),
                            preferred_element_type=jnp.float32)
        m0 = jnp.max(s, axis=1, keepdims=True)
        e0 = jnp.exp(s - m0)
        a0 = e0 / jnp.sum(e0, axis=1, keepdims=True)
        m1 = jnp.max(s, axis=0, keepdims=True)
        e1 = jnp.exp(s - m1)
        a1 = e1 / jnp.sum(e1, axis=0, keepdims=True)
        att0 = jnp.dot(a0, f1, preferred_element_type=jnp.float32)
        att1 = lax.dot_general(a1, f0, (((0,), (0,)), ((), ())),
                               preferred_element_type=jnp.float32)
        mu0 = f0 - att0
        mu1 = f1 - att1
        h0 = jax.nn.relu(
            jnp.dot(f0, wh_t, preferred_element_type=jnp.float32)
            + jnp.dot(mu0, wh_b, preferred_element_type=jnp.float32) + bh)
        h1 = jax.nn.relu(
            jnp.dot(f1, wh_t, preferred_element_type=jnp.float32)
            + jnp.dot(mu1, wh_b, preferred_element_type=jnp.float32) + bh)
        f0 = f0 + jnp.dot(h0, wo, preferred_element_type=jnp.float32) + bo
        f1 = f1 + jnp.dot(h1, wo, preferred_element_type=jnp.float32) + bo
    o0_ref[0] = f0
    o1_ref[0] = f1


def _cross_call(f0, f1, wh, bh, wo, bo):
    return pl.pallas_call(
        _cross_body,
        grid=(B,),
        in_specs=[
            pl.BlockSpec((1, NP, DIM), lambda b: (b, 0, 0)),
            pl.BlockSpec((1, NP, DIM), lambda b: (b, 0, 0)),
            pl.BlockSpec((NPROP, 2 * DIM, HID), lambda b: (0, 0, 0)),
            pl.BlockSpec((NPROP, 1, HID), lambda b: (0, 0, 0)),
            pl.BlockSpec((NPROP, HID, DIM), lambda b: (0, 0, 0)),
            pl.BlockSpec((NPROP, 1, DIM), lambda b: (0, 0, 0)),
        ],
        out_specs=[
            pl.BlockSpec((1, NP, DIM), lambda b: (b, 0, 0)),
            pl.BlockSpec((1, NP, DIM), lambda b: (b, 0, 0)),
        ],
        out_shape=[
            jax.ShapeDtypeStruct((B, NP, DIM), jnp.float32),
            jax.ShapeDtypeStruct((B, NP, DIM), jnp.float32),
        ],
        interpret=_INTERPRET,
    )(f0, f1, wh, bh, wo, bo)


def kernel(input_xyz, coord_xyz, input_f, coord_f, Wf, bf, Ws, bs,
           gamma, beta, Wh, bh, Wo, bo):
    x0 = input_f.reshape(NNODE, DIM)
    x1 = coord_f.reshape(NNODE, DIM)
    table = jnp.concatenate([x0, x1], axis=0)

    wft, wfb = Wf[:DIM], Wf[DIM:]
    wst, wsb = Ws[:DIM], Ws[DIM:]
    bf2 = bf.reshape(1, DIM)
    bs2 = bs.reshape(1, DIM)
    nt = NNODE // _TN

    fs = []
    pad = jnp.zeros((B, NP, 5), jnp.float32)
    for g, (xyz, x) in enumerate(((input_xyz, x0), (coord_xyz, x1))):
        xyzp = jnp.concatenate([xyz, pad], axis=-1)
        xyzt = jnp.swapaxes(xyzp, 1, 2)
        idx = _knn_call(xyzp, xyzt, g * NNODE)
        xg = _gather_call(idx.reshape(-1), table)
        agg = _msg_call(
            x.reshape(nt, _TN, DIM),
            xg.reshape(B, K, NP, DIM),
            wft, wfb, wst, wsb, bf2, bs2,
        )
        fs.append(_norm_call(agg.reshape(NNODE, DIM), x,
                             gamma.reshape(1, DIM), beta.reshape(1, DIM)))

    o0, o1 = _cross_call(fs[0].reshape(B, NP, DIM), fs[1].reshape(B, NP, DIM),
                         Wh, bh.reshape(NPROP, 1, HID),
                         Wo, bo.reshape(NPROP, 1, DIM))
    return o0.reshape(-1, DIM), o1.reshape(-1, DIM)


# speedup vs baseline: 7.9024x; 1.0076x over previous
import functools

import jax
import jax.numpy as jnp
from jax import lax
from jax.experimental import pallas as pl
from jax.experimental.pallas import tpu as pltpu
from jax.experimental.pallas import tpu_sc as plsc

B, NP, DIM = 4, 1024, 128
K = 16
HID, NPROP = 64, 5
NG = 2
NNODE = B * NP
NEDGE = NNODE * K

_INTERPRET = False


def _knn_body(xyz_ref, xyzt_ref, out_ref, *, gbase):
    d2 = jnp.zeros((NP, NP), jnp.float32)
    for c in range(3):
        col = xyz_ref[0, :, c : c + 1]
        row = xyzt_ref[0, c : c + 1, :]
        diff = col - row
        d2 = d2 + diff * diff
    ii = lax.broadcasted_iota(jnp.int32, (NP, NP), 0)
    jj = lax.broadcasted_iota(jnp.int32, (NP, NP), 1)
    d2 = jnp.where(ii == jj, d2 + 1e10, d2)
    rows = []
    m = jnp.min(d2, axis=0, keepdims=True)
    for r in range(K):
        cand = jnp.where(d2 == m, ii, NP)
        sel = jnp.min(cand, axis=0, keepdims=True)
        rows.append(sel)
        if r + 1 < K:
            d2 = jnp.where(ii == sel, jnp.float32(1e30), d2)
            m = jnp.min(d2, axis=0, keepdims=True)
    b = pl.program_id(0)
    base = gbase + b * NP
    out_ref[0] = jnp.concatenate(rows, axis=0) + base


def _knn_call(xyzp, xyzt, gbase):
    return pl.pallas_call(
        functools.partial(_knn_body, gbase=gbase),
        grid=(B,),
        in_specs=[
            pl.BlockSpec((1, NP, 8), lambda b: (b, 0, 0)),
            pl.BlockSpec((1, 8, NP), lambda b: (b, 0, 0)),
        ],
        out_specs=pl.BlockSpec((1, K, NP), lambda b: (b, 0, 0)),
        out_shape=jax.ShapeDtypeStruct((B, K, NP), jnp.int32),
        interpret=_INTERPRET,
    )(xyzp, xyzt)


_NC, _NS = 2, 16
_NW = _NC * _NS
_CHUNK = 128


def _make_gather(n_rows, d):
    per_w = n_rows // _NW
    n_chunks = per_w // _CHUNK
    mesh = plsc.VectorSubcoreMesh(core_axis_name="c", subcore_axis_name="s")

    @functools.partial(
        pl.kernel,
        mesh=mesh,
        out_type=jax.ShapeDtypeStruct((n_rows, d), jnp.float32),
        scratch_types=[
            pltpu.VMEM((per_w,), jnp.int32),
            pltpu.VMEM((_CHUNK, d), jnp.float32),
            pltpu.VMEM((_CHUNK, d), jnp.float32),
            pltpu.SemaphoreType.DMA,
            pltpu.SemaphoreType.DMA,
            pltpu.SemaphoreType.DMA,
            pltpu.SemaphoreType.DMA,
        ],
    )
    def gk(idx_hbm, table_hbm, out_hbm, idx_v, b0, b1, sg0, sg1, ss0, ss1):
        wid = lax.axis_index("s") * _NC + lax.axis_index("c")
        base = wid * per_w
        bufs = (b0, b1)
        gsem = (sg0, sg1)
        ssem = (ss0, ss1)
        pltpu.sync_copy(idx_hbm.at[pl.ds(base, per_w)], idx_v)
        cp_g = {}
        cp_s = {}
        cp_g[0] = pltpu.async_copy(
            table_hbm.at[idx_v.at[pl.ds(0, _CHUNK)]], bufs[0], gsem[0])
        for c in range(n_chunks):
            cp_g[c].wait()
            cp_s[c] = pltpu.async_copy(
                bufs[c % 2], out_hbm.at[pl.ds(base + c * _CHUNK, _CHUNK)],
                ssem[c % 2])
            if c + 1 < n_chunks:
                if c >= 1:
                    cp_s[c - 1].wait()
                cp_g[c + 1] = pltpu.async_copy(
                    table_hbm.at[idx_v.at[pl.ds((c + 1) * _CHUNK, _CHUNK)]],
                    bufs[(c + 1) % 2], gsem[(c + 1) % 2])
        cp_s[n_chunks - 1].wait()
        if n_chunks >= 2:
            cp_s[n_chunks - 2].wait()

    return gk


_gather_kernel = None


def _gather_call(src, table):
    global _gather_kernel
    if _gather_kernel is None:
        _gather_kernel = _make_gather(NEDGE, DIM)
    return _gather_kernel(src, table)


_TN = 256


def _msg_body(xd_ref, xg_ref, wft_ref, wfb_ref, wst_ref, wsb_ref,
              bf_ref, bs_ref, out_ref):
    xd = xd_ref[0]
    xg = xg_ref[0, :, :, :].reshape(K * _TN, DIM)
    p = jnp.dot(xd, wft_ref[...], preferred_element_type=jnp.float32) + bf_ref[...]
    r = jnp.dot(xd, wst_ref[...], preferred_element_type=jnp.float32) + bs_ref[...]
    q = jnp.dot(xg, wfb_ref[...], preferred_element_type=jnp.float32)
    s = jnp.dot(xg, wsb_ref[...], preferred_element_type=jnp.float32)
    a = q.reshape(K, _TN, DIM) + p.reshape(1, _TN, DIM)
    t = s.reshape(K, _TN, DIM) + r.reshape(1, _TN, DIM)
    sig = 1.0 / (1.0 + jnp.exp(-a))
    sp = jnp.maximum(t, 0.0) + jnp.log1p(jnp.exp(-jnp.abs(t)))
    out_ref[0] = jnp.max(sig * sp, axis=0)


def _msg_call(xd, xg, wft, wfb, wst, wsb, bf, bs):
    nt = NNODE // _TN
    npb = NP // _TN
    return pl.pallas_call(
        _msg_body,
        grid=(nt,),
        in_specs=[
            pl.BlockSpec((1, _TN, DIM), lambda t: (t, 0, 0)),
            pl.BlockSpec((1, K, _TN, DIM),
                         lambda t: (t // npb, 0, t % npb, 0)),
            pl.BlockSpec((DIM, DIM), lambda t: (0, 0)),
            pl.BlockSpec((DIM, DIM), lambda t: (0, 0)),
            pl.BlockSpec((DIM, DIM), lambda t: (0, 0)),
            pl.BlockSpec((DIM, DIM), lambda t: (0, 0)),
            pl.BlockSpec((1, DIM), lambda t: (0, 0)),
            pl.BlockSpec((1, DIM), lambda t: (0, 0)),
        ],
        out_specs=pl.BlockSpec((1, _TN, DIM), lambda t: (t, 0, 0)),
        out_shape=jax.ShapeDtypeStruct((nt, _TN, DIM), jnp.float32),
        interpret=_INTERPRET,
    )(xd, xg, wft, wfb, wst, wsb, bf, bs)


def _norm_body(agg_ref, x_ref, gamma_ref, beta_ref, out_ref):
    a = agg_ref[...]
    mu = jnp.mean(a, axis=0, keepdims=True)
    var = jnp.mean((a - mu) ** 2, axis=0, keepdims=True)
    nrm = (a - mu) / jnp.sqrt(var + 1e-5) * gamma_ref[...] + beta_ref[...]
    out_ref[...] = x_ref[...] + nrm


def _norm_call(agg, x, gamma, beta):
    return pl.pallas_call(
        _norm_body,
        in_specs=[
            pl.BlockSpec((NNODE, DIM), lambda: (0, 0)),
            pl.BlockSpec((NNODE, DIM), lambda: (0, 0)),
            pl.BlockSpec((1, DIM), lambda: (0, 0)),
            pl.BlockSpec((1, DIM), lambda: (0, 0)),
        ],
        out_specs=pl.BlockSpec((NNODE, DIM), lambda: (0, 0)),
        out_shape=jax.ShapeDtypeStruct((NNODE, DIM), jnp.float32),
        interpret=_INTERPRET,
    )(agg, x, gamma, beta)


def _cross_body(agg0_ref, ab0_ref, xb0_ref, agg1_ref, ab1_ref, xb1_ref,
                gamma_ref, beta_ref,
                wh_ref, bh_ref, wo_ref, bo_ref, o0_ref, o1_ref):
    fs = []
    for agg_ref, ab_ref, xb_ref in ((agg0_ref, ab0_ref, xb0_ref),
                                    (agg1_ref, ab1_ref, xb1_ref)):
        a = agg_ref[...]
        mu = jnp.mean(a, axis=0, keepdims=True)
        var = jnp.mean((a - mu) ** 2, axis=0, keepdims=True)
        nrm = (ab_ref[0] - mu) / jnp.sqrt(var + 1e-5) * gamma_ref[...] \
            + beta_ref[...]
        fs.append(xb_ref[0] + nrm)
    f0, f1 = fs
    for l in range(NPROP):
        wh_t = wh_ref[l, :DIM, :]
        wh_b = wh_ref[l, DIM:, :]
        bh = bh_ref[l]
        wo = wo_ref[l]
        bo = bo_ref[l]
        s = lax.dot_general(f0, f1, (((1,), (1,)), ((), ())),
                            preferred_element_type=jnp.float32)
        m0 = jnp.max(s, axis=1, keepdims=True)
        e0 = jnp.exp(s - m0)
        a0 = e0 / jnp.sum(e0, axis=1, keepdims=True)
        m1 = jnp.max(s, axis=0, keepdims=True)
        e1 = jnp.exp(s - m1)
        a1 = e1 / jnp.sum(e1, axis=0, keepdims=True)
        att0 = jnp.dot(a0, f1, preferred_element_type=jnp.float32)
        att1 = lax.dot_general(a1, f0, (((0,), (0,)), ((), ())),
                               preferred_element_type=jnp.float32)
        mu0 = f0 - att0
        mu1 = f1 - att1
        h0 = jax.nn.relu(
            jnp.dot(f0, wh_t, preferred_element_type=jnp.float32)
            + jnp.dot(mu0, wh_b, preferred_element_type=jnp.float32) + bh)
        h1 = jax.nn.relu(
            jnp.dot(f1, wh_t, preferred_element_type=jnp.float32)
            + jnp.dot(mu1, wh_b, preferred_element_type=jnp.float32) + bh)
        f0 = f0 + jnp.dot(h0, wo, preferred_element_type=jnp.float32) + bo
        f1 = f1 + jnp.dot(h1, wo, preferred_element_type=jnp.float32) + bo
    o0_ref[0] = f0
    o1_ref[0] = f1


def _cross_call(agg0, x0, agg1, x1, gamma, beta, wh, bh, wo, bo):
    full = pl.BlockSpec((NNODE, DIM), lambda b: (0, 0))
    batch = pl.BlockSpec((1, NP, DIM), lambda b: (b, 0, 0))
    return pl.pallas_call(
        _cross_body,
        grid=(B,),
        in_specs=[
            full, batch, batch,
            full, batch, batch,
            pl.BlockSpec((1, DIM), lambda b: (0, 0)),
            pl.BlockSpec((1, DIM), lambda b: (0, 0)),
            pl.BlockSpec((NPROP, 2 * DIM, HID), lambda b: (0, 0, 0)),
            pl.BlockSpec((NPROP, 1, HID), lambda b: (0, 0, 0)),
            pl.BlockSpec((NPROP, HID, DIM), lambda b: (0, 0, 0)),
            pl.BlockSpec((NPROP, 1, DIM), lambda b: (0, 0, 0)),
        ],
        out_specs=[
            pl.BlockSpec((1, NP, DIM), lambda b: (b, 0, 0)),
            pl.BlockSpec((1, NP, DIM), lambda b: (b, 0, 0)),
        ],
        out_shape=[
            jax.ShapeDtypeStruct((B, NP, DIM), jnp.float32),
            jax.ShapeDtypeStruct((B, NP, DIM), jnp.float32),
        ],
        interpret=_INTERPRET,
    )(agg0, agg0.reshape(B, NP, DIM), x0.reshape(B, NP, DIM),
      agg1, agg1.reshape(B, NP, DIM), x1.reshape(B, NP, DIM),
      gamma, beta, wh, bh, wo, bo)


def kernel(input_xyz, coord_xyz, input_f, coord_f, Wf, bf, Ws, bs,
           gamma, beta, Wh, bh, Wo, bo):
    x0 = input_f.reshape(NNODE, DIM)
    x1 = coord_f.reshape(NNODE, DIM)
    table = jnp.concatenate([x0, x1], axis=0)

    wft, wfb = Wf[:DIM], Wf[DIM:]
    wst, wsb = Ws[:DIM], Ws[DIM:]
    bf2 = bf.reshape(1, DIM)
    bs2 = bs.reshape(1, DIM)
    nt = NNODE // _TN

    aggs = []
    pad = jnp.zeros((B, NP, 5), jnp.float32)
    for g, (xyz, x) in enumerate(((input_xyz, x0), (coord_xyz, x1))):
        xyzp = jnp.concatenate([xyz, pad], axis=-1)
        xyzt = jnp.swapaxes(xyzp, 1, 2)
        idx = _knn_call(xyzp, xyzt, g * NNODE)
        xg = _gather_call(idx.reshape(-1), table)
        agg = _msg_call(
            x.reshape(nt, _TN, DIM),
            xg.reshape(B, K, NP, DIM),
            wft, wfb, wst, wsb, bf2, bs2,
        )
        aggs.append(agg.reshape(NNODE, DIM))

    o0, o1 = _cross_call(aggs[0], x0, aggs[1], x1,
                         gamma.reshape(1, DIM), beta.reshape(1, DIM),
                         Wh, bh.reshape(NPROP, 1, HID),
                         Wo, bo.reshape(NPROP, 1, DIM))
    return o0.reshape(-1, DIM), o1.reshape(-1, DIM)


# speedup vs baseline: 7.9740x; 1.0091x over previous
import functools

import jax
import jax.numpy as jnp
from jax import lax
from jax.experimental import pallas as pl
from jax.experimental.pallas import tpu as pltpu
from jax.experimental.pallas import tpu_sc as plsc

B, NP, DIM = 4, 1024, 128
K = 16
HID, NPROP = 64, 5
NG = 2
NNODE = B * NP
NEDGE = NNODE * K

_INTERPRET = False


def _knn_body(xyz_ref, xyzt_ref, out_ref):
    d2 = jnp.zeros((NP, NP), jnp.float32)
    for c in range(3):
        col = xyz_ref[0, :, c : c + 1]
        row = xyzt_ref[0, c : c + 1, :]
        diff = col - row
        d2 = d2 + diff * diff
    ii = lax.broadcasted_iota(jnp.int32, (NP, NP), 0)
    jj = lax.broadcasted_iota(jnp.int32, (NP, NP), 1)
    d2 = jnp.where(ii == jj, d2 + 1e10, d2)
    rows = []
    m = jnp.min(d2, axis=0, keepdims=True)
    for r in range(K):
        cand = jnp.where(d2 == m, ii, NP)
        sel = jnp.min(cand, axis=0, keepdims=True)
        rows.append(sel)
        if r + 1 < K:
            d2 = jnp.where(ii == sel, jnp.float32(1e30), d2)
            m = jnp.min(d2, axis=0, keepdims=True)
    b = pl.program_id(0)
    out_ref[0] = jnp.concatenate(rows, axis=0) + b * NP


def _knn_call(xyzp, xyzt):
    return pl.pallas_call(
        _knn_body,
        grid=(B,),
        in_specs=[
            pl.BlockSpec((1, NP, 8), lambda b: (b, 0, 0)),
            pl.BlockSpec((1, 8, NP), lambda b: (b, 0, 0)),
        ],
        out_specs=pl.BlockSpec((1, K, NP), lambda b: (b, 0, 0)),
        out_shape=jax.ShapeDtypeStruct((B, K, NP), jnp.int32),
        interpret=_INTERPRET,
    )(xyzp, xyzt)


_NC, _NS = 2, 16
_NW = _NC * _NS
_CHUNK = 128


def _make_gather(n_rows, d):
    per_w = n_rows // _NW
    n_chunks = per_w // _CHUNK
    mesh = plsc.VectorSubcoreMesh(core_axis_name="c", subcore_axis_name="s")

    @functools.partial(
        pl.kernel,
        mesh=mesh,
        out_type=jax.ShapeDtypeStruct((n_rows, d), jnp.float32),
        scratch_types=[
            pltpu.VMEM((per_w,), jnp.int32),
            pltpu.VMEM((_CHUNK, d), jnp.float32),
            pltpu.VMEM((_CHUNK, d), jnp.float32),
            pltpu.SemaphoreType.DMA,
            pltpu.SemaphoreType.DMA,
            pltpu.SemaphoreType.DMA,
            pltpu.SemaphoreType.DMA,
        ],
    )
    def gk(idx_hbm, table_hbm, out_hbm, idx_v, b0, b1, sg0, sg1, ss0, ss1):
        wid = lax.axis_index("s") * _NC + lax.axis_index("c")
        base = wid * per_w
        bufs = (b0, b1)
        gsem = (sg0, sg1)
        ssem = (ss0, ss1)
        pltpu.sync_copy(idx_hbm.at[pl.ds(base, per_w)], idx_v)
        cp_g = {}
        cp_s = {}
        cp_g[0] = pltpu.async_copy(
            table_hbm.at[idx_v.at[pl.ds(0, _CHUNK)]], bufs[0], gsem[0])
        for c in range(n_chunks):
            cp_g[c].wait()
            cp_s[c] = pltpu.async_copy(
                bufs[c % 2], out_hbm.at[pl.ds(base + c * _CHUNK, _CHUNK)],
                ssem[c % 2])
            if c + 1 < n_chunks:
                if c >= 1:
                    cp_s[c - 1].wait()
                cp_g[c + 1] = pltpu.async_copy(
                    table_hbm.at[idx_v.at[pl.ds((c + 1) * _CHUNK, _CHUNK)]],
                    bufs[(c + 1) % 2], gsem[(c + 1) % 2])
        cp_s[n_chunks - 1].wait()
        if n_chunks >= 2:
            cp_s[n_chunks - 2].wait()

    return gk


_gather_kernel = None


def _gather_call(src, table):
    global _gather_kernel
    if _gather_kernel is None:
        _gather_kernel = _make_gather(NEDGE, DIM)
    return _gather_kernel(src, table)


_TN = 256


def _msg_body(xd_ref, xg_ref, wft_ref, wfb_ref, wst_ref, wsb_ref,
              bf_ref, bs_ref, out_ref):
    xd = xd_ref[0]
    xg = xg_ref[0, :, :, :].reshape(K * _TN, DIM)
    p = jnp.dot(xd, wft_ref[...], preferred_element_type=jnp.float32) + bf_ref[...]
    r = jnp.dot(xd, wst_ref[...], preferred_element_type=jnp.float32) + bs_ref[...]
    q = jnp.dot(xg, wfb_ref[...], preferred_element_type=jnp.float32)
    s = jnp.dot(xg, wsb_ref[...], preferred_element_type=jnp.float32)
    a = q.reshape(K, _TN, DIM) + p.reshape(1, _TN, DIM)
    t = s.reshape(K, _TN, DIM) + r.reshape(1, _TN, DIM)
    sig = 1.0 / (1.0 + jnp.exp(-a))
    sp = jnp.maximum(t, 0.0) + jnp.log1p(jnp.exp(-jnp.abs(t)))
    out_ref[0] = jnp.max(sig * sp, axis=0)


def _msg_call(xd, xg, wft, wfb, wst, wsb, bf, bs):
    nt = NNODE // _TN
    npb = NP // _TN
    return pl.pallas_call(
        _msg_body,
        grid=(nt,),
        in_specs=[
            pl.BlockSpec((1, _TN, DIM), lambda t: (t, 0, 0)),
            pl.BlockSpec((1, K, _TN, DIM),
                         lambda t: (t // npb, 0, t % npb, 0)),
            pl.BlockSpec((DIM, DIM), lambda t: (0, 0)),
            pl.BlockSpec((DIM, DIM), lambda t: (0, 0)),
            pl.BlockSpec((DIM, DIM), lambda t: (0, 0)),
            pl.BlockSpec((DIM, DIM), lambda t: (0, 0)),
            pl.BlockSpec((1, DIM), lambda t: (0, 0)),
            pl.BlockSpec((1, DIM), lambda t: (0, 0)),
        ],
        out_specs=pl.BlockSpec((1, _TN, DIM), lambda t: (t, 0, 0)),
        out_shape=jax.ShapeDtypeStruct((nt, _TN, DIM), jnp.float32),
        interpret=_INTERPRET,
    )(xd, xg, wft, wfb, wst, wsb, bf, bs)


def _norm_body(agg_ref, x_ref, gamma_ref, beta_ref, out_ref):
    a = agg_ref[...]
    mu = jnp.mean(a, axis=0, keepdims=True)
    var = jnp.mean((a - mu) ** 2, axis=0, keepdims=True)
    nrm = (a - mu) / jnp.sqrt(var + 1e-5) * gamma_ref[...] + beta_ref[...]
    out_ref[...] = x_ref[...] + nrm


def _norm_call(agg, x, gamma, beta):
    return pl.pallas_call(
        _norm_body,
        in_specs=[
            pl.BlockSpec((NNODE, DIM), lambda: (0, 0)),
            pl.BlockSpec((NNODE, DIM), lambda: (0, 0)),
            pl.BlockSpec((1, DIM), lambda: (0, 0)),
            pl.BlockSpec((1, DIM), lambda: (0, 0)),
        ],
        out_specs=pl.BlockSpec((NNODE, DIM), lambda: (0, 0)),
        out_shape=jax.ShapeDtypeStruct((NNODE, DIM), jnp.float32),
        interpret=_INTERPRET,
    )(agg, x, gamma, beta)


def _cross_body(agg0_ref, ab0_ref, xb0_ref, agg1_ref, ab1_ref, xb1_ref,
                gamma_ref, beta_ref,
                wh_ref, bh_ref, wo_ref, bo_ref, o0_ref, o1_ref):
    fs = []
    for agg_ref, ab_ref, xb_ref in ((agg0_ref, ab0_ref, xb0_ref),
                                    (agg1_ref, ab1_ref, xb1_ref)):
        a = agg_ref[...]
        mu = jnp.mean(a, axis=0, keepdims=True)
        var = jnp.mean((a - mu) ** 2, axis=0, keepdims=True)
        nrm = (ab_ref[0] - mu) / jnp.sqrt(var + 1e-5) * gamma_ref[...] \
            + beta_ref[...]
        fs.append(xb_ref[0] + nrm)
    f0, f1 = fs
    for l in range(NPROP):
        wh_t = wh_ref[l, :DIM, :]
        wh_b = wh_ref[l, DIM:, :]
        bh = bh_ref[l]
        wo = wo_ref[l]
        bo = bo_ref[l]
        s = lax.dot_general(f0, f1, (((1,), (1,)), ((), ())),
                            preferred_element_type=jnp.float32)
        m0 = jnp.max(s, axis=1, keepdims=True)
        e0 = jnp.exp(s - m0)
        a0 = e0 / jnp.sum(e0, axis=1, keepdims=True)
        m1 = jnp.max(s, axis=0, keepdims=True)
        e1 = jnp.exp(s - m1)
        a1 = e1 / jnp.sum(e1, axis=0, keepdims=True)
        att0 = jnp.dot(a0, f1, preferred_element_type=jnp.float32)
        att1 = lax.dot_general(a1, f0, (((0,), (0,)), ((), ())),
                               preferred_element_type=jnp.float32)
        mu0 = f0 - att0
        mu1 = f1 - att1
        h0 = jax.nn.relu(
            jnp.dot(f0, wh_t, preferred_element_type=jnp.float32)
            + jnp.dot(mu0, wh_b, preferred_element_type=jnp.float32) + bh)
        h1 = jax.nn.relu(
            jnp.dot(f1, wh_t, preferred_element_type=jnp.float32)
            + jnp.dot(mu1, wh_b, preferred_element_type=jnp.float32) + bh)
        f0 = f0 + jnp.dot(h0, wo, preferred_element_type=jnp.float32) + bo
        f1 = f1 + jnp.dot(h1, wo, preferred_element_type=jnp.float32) + bo
    o0_ref[0] = f0
    o1_ref[0] = f1


def _cross_call(agg0, x0, agg1, x1, gamma, beta, wh, bh, wo, bo):
    full = pl.BlockSpec((NNODE, DIM), lambda b: (0, 0))
    batch = pl.BlockSpec((1, NP, DIM), lambda b: (b, 0, 0))
    return pl.pallas_call(
        _cross_body,
        grid=(B,),
        in_specs=[
            full, batch, batch,
            full, batch, batch,
            pl.BlockSpec((1, DIM), lambda b: (0, 0)),
            pl.BlockSpec((1, DIM), lambda b: (0, 0)),
            pl.BlockSpec((NPROP, 2 * DIM, HID), lambda b: (0, 0, 0)),
            pl.BlockSpec((NPROP, 1, HID), lambda b: (0, 0, 0)),
            pl.BlockSpec((NPROP, HID, DIM), lambda b: (0, 0, 0)),
            pl.BlockSpec((NPROP, 1, DIM), lambda b: (0, 0, 0)),
        ],
        out_specs=[
            pl.BlockSpec((1, NP, DIM), lambda b: (b, 0, 0)),
            pl.BlockSpec((1, NP, DIM), lambda b: (b, 0, 0)),
        ],
        out_shape=[
            jax.ShapeDtypeStruct((B, NP, DIM), jnp.float32),
            jax.ShapeDtypeStruct((B, NP, DIM), jnp.float32),
        ],
        interpret=_INTERPRET,
    )(agg0, agg0.reshape(B, NP, DIM), x0.reshape(B, NP, DIM),
      agg1, agg1.reshape(B, NP, DIM), x1.reshape(B, NP, DIM),
      gamma, beta, wh, bh, wo, bo)


def kernel(input_xyz, coord_xyz, input_f, coord_f, Wf, bf, Ws, bs,
           gamma, beta, Wh, bh, Wo, bo):
    x0 = input_f.reshape(NNODE, DIM)
    x1 = coord_f.reshape(NNODE, DIM)

    wft, wfb = Wf[:DIM], Wf[DIM:]
    wst, wsb = Ws[:DIM], Ws[DIM:]
    bf2 = bf.reshape(1, DIM)
    bs2 = bs.reshape(1, DIM)
    nt = NNODE // _TN

    aggs = []
    pad = jnp.zeros((B, NP, 5), jnp.float32)
    for xyz, x in ((input_xyz, x0), (coord_xyz, x1)):
        xyzp = jnp.concatenate([xyz, pad], axis=-1)
        xyzt = jnp.swapaxes(xyzp, 1, 2)
        idx = _knn_call(xyzp, xyzt)
        xg = _gather_call(idx.reshape(-1), x)
        agg = _msg_call(
            x.reshape(nt, _TN, DIM),
            xg.reshape(B, K, NP, DIM),
            wft, wfb, wst, wsb, bf2, bs2,
        )
        aggs.append(agg.reshape(NNODE, DIM))

    o0, o1 = _cross_call(aggs[0], x0, aggs[1], x1,
                         gamma.reshape(1, DIM), beta.reshape(1, DIM),
                         Wh, bh.reshape(NPROP, 1, HID),
                         Wo, bo.reshape(NPROP, 1, DIM))
    return o0.reshape(-1, DIM), o1.reshape(-1, DIM)


# speedup vs baseline: 8.2851x; 1.0390x over previous
import functools

import jax
import jax.numpy as jnp
from jax import lax
from jax.experimental import pallas as pl
from jax.experimental.pallas import tpu as pltpu
from jax.experimental.pallas import tpu_sc as plsc

B, NP, DIM = 4, 1024, 128
K = 16
HID, NPROP = 64, 5
NG = 2
NNODE = B * NP
NEDGE = NNODE * K

_INTERPRET = False


def _knn_body(xyz_ref, xyzt_ref, out_ref):
    d2 = jnp.zeros((NP, NP), jnp.float32)
    for c in range(3):
        col = xyz_ref[0, :, c : c + 1]
        row = xyzt_ref[0, c : c + 1, :]
        diff = col - row
        d2 = d2 + diff * diff
    ii = lax.broadcasted_iota(jnp.int32, (NP, NP), 0)
    jj = lax.broadcasted_iota(jnp.int32, (NP, NP), 1)
    d2 = jnp.where(ii == jj, d2 + 1e10, d2)
    rows = []
    m = jnp.min(d2, axis=0, keepdims=True)
    for r in range(K):
        cand = jnp.where(d2 == m, ii, NP)
        sel = jnp.min(cand, axis=0, keepdims=True)
        rows.append(sel)
        if r + 1 < K:
            d2 = jnp.where(ii == sel, jnp.float32(1e30), d2)
            m = jnp.min(d2, axis=0, keepdims=True)
    b = pl.program_id(0)
    out_ref[0] = jnp.concatenate(rows, axis=0) + b * NP


def _knn_call(xyzp, xyzt):
    return pl.pallas_call(
        _knn_body,
        grid=(B,),
        in_specs=[
            pl.BlockSpec((1, NP, 8), lambda b: (b, 0, 0)),
            pl.BlockSpec((1, 8, NP), lambda b: (b, 0, 0)),
        ],
        out_specs=pl.BlockSpec((1, K, NP), lambda b: (b, 0, 0)),
        out_shape=jax.ShapeDtypeStruct((B, K, NP), jnp.int32),
        interpret=_INTERPRET,
    )(xyzp, xyzt)


_NC, _NS = 2, 16
_NW = _NC * _NS
_CHUNK = 128


def _make_gather(n_rows, d):
    per_w = n_rows // _NW
    n_chunks = per_w // _CHUNK
    mesh = plsc.VectorSubcoreMesh(core_axis_name="c", subcore_axis_name="s")

    @functools.partial(
        pl.kernel,
        mesh=mesh,
        out_type=jax.ShapeDtypeStruct((n_rows, d), jnp.float32),
        scratch_types=[
            pltpu.VMEM((per_w,), jnp.int32),
            pltpu.VMEM((_CHUNK, d), jnp.float32),
            pltpu.VMEM((_CHUNK, d), jnp.float32),
            pltpu.SemaphoreType.DMA,
            pltpu.SemaphoreType.DMA,
            pltpu.SemaphoreType.DMA,
            pltpu.SemaphoreType.DMA,
        ],
    )
    def gk(idx_hbm, table_hbm, out_hbm, idx_v, b0, b1, sg0, sg1, ss0, ss1):
        wid = lax.axis_index("s") * _NC + lax.axis_index("c")
        base = wid * per_w
        bufs = (b0, b1)
        gsem = (sg0, sg1)
        ssem = (ss0, ss1)
        pltpu.sync_copy(idx_hbm.at[pl.ds(base, per_w)], idx_v)
        cp_g = {}
        cp_s = {}
        cp_g[0] = pltpu.async_copy(
            table_hbm.at[idx_v.at[pl.ds(0, _CHUNK)]], bufs[0], gsem[0])
        for c in range(n_chunks):
            cp_g[c].wait()
            cp_s[c] = pltpu.async_copy(
                bufs[c % 2], out_hbm.at[pl.ds(base + c * _CHUNK, _CHUNK)],
                ssem[c % 2])
            if c + 1 < n_chunks:
                if c >= 1:
                    cp_s[c - 1].wait()
                cp_g[c + 1] = pltpu.async_copy(
                    table_hbm.at[idx_v.at[pl.ds((c + 1) * _CHUNK, _CHUNK)]],
                    bufs[(c + 1) % 2], gsem[(c + 1) % 2])
        cp_s[n_chunks - 1].wait()
        if n_chunks >= 2:
            cp_s[n_chunks - 2].wait()

    return gk


_gather_kernel = None


def _gather_call(src, table):
    global _gather_kernel
    if _gather_kernel is None:
        _gather_kernel = _make_gather(NEDGE, DIM)
    return _gather_kernel(src, table)


_TN = 512


def _msg_body(xd_ref, xg_ref, wft_ref, wfb_ref, wst_ref, wsb_ref,
              bf_ref, bs_ref, out_ref):
    xd = xd_ref[0]
    xg = xg_ref[0, :, :, :].reshape(K * _TN, DIM)
    p = jnp.dot(xd, wft_ref[...], preferred_element_type=jnp.float32) + bf_ref[...]
    r = jnp.dot(xd, wst_ref[...], preferred_element_type=jnp.float32) + bs_ref[...]
    q = jnp.dot(xg, wfb_ref[...], preferred_element_type=jnp.float32)
    s = jnp.dot(xg, wsb_ref[...], preferred_element_type=jnp.float32)
    a = q.reshape(K, _TN, DIM) + p.reshape(1, _TN, DIM)
    t = s.reshape(K, _TN, DIM) + r.reshape(1, _TN, DIM)
    sig = 1.0 / (1.0 + jnp.exp(-a))
    sp = jnp.maximum(t, 0.0) + jnp.log1p(jnp.exp(-jnp.abs(t)))
    out_ref[0] = jnp.max(sig * sp, axis=0)


def _msg_call(xd, xg, wft, wfb, wst, wsb, bf, bs):
    nt = NNODE // _TN
    npb = NP // _TN
    return pl.pallas_call(
        _msg_body,
        grid=(nt,),
        in_specs=[
            pl.BlockSpec((1, _TN, DIM), lambda t: (t, 0, 0)),
            pl.BlockSpec((1, K, _TN, DIM),
                         lambda t: (t // npb, 0, t % npb, 0)),
            pl.BlockSpec((DIM, DIM), lambda t: (0, 0)),
            pl.BlockSpec((DIM, DIM), lambda t: (0, 0)),
            pl.BlockSpec((DIM, DIM), lambda t: (0, 0)),
            pl.BlockSpec((DIM, DIM), lambda t: (0, 0)),
            pl.BlockSpec((1, DIM), lambda t: (0, 0)),
            pl.BlockSpec((1, DIM), lambda t: (0, 0)),
        ],
        out_specs=pl.BlockSpec((1, _TN, DIM), lambda t: (t, 0, 0)),
        out_shape=jax.ShapeDtypeStruct((nt, _TN, DIM), jnp.float32),
        interpret=_INTERPRET,
    )(xd, xg, wft, wfb, wst, wsb, bf, bs)


def _norm_body(agg_ref, x_ref, gamma_ref, beta_ref, out_ref):
    a = agg_ref[...]
    mu = jnp.mean(a, axis=0, keepdims=True)
    var = jnp.mean((a - mu) ** 2, axis=0, keepdims=True)
    nrm = (a - mu) / jnp.sqrt(var + 1e-5) * gamma_ref[...] + beta_ref[...]
    out_ref[...] = x_ref[...] + nrm


def _norm_call(agg, x, gamma, beta):
    return pl.pallas_call(
        _norm_body,
        in_specs=[
            pl.BlockSpec((NNODE, DIM), lambda: (0, 0)),
            pl.BlockSpec((NNODE, DIM), lambda: (0, 0)),
            pl.BlockSpec((1, DIM), lambda: (0, 0)),
            pl.BlockSpec((1, DIM), lambda: (0, 0)),
        ],
        out_specs=pl.BlockSpec((NNODE, DIM), lambda: (0, 0)),
        out_shape=jax.ShapeDtypeStruct((NNODE, DIM), jnp.float32),
        interpret=_INTERPRET,
    )(agg, x, gamma, beta)


def _cross_body(agg0_ref, ab0_ref, xb0_ref, agg1_ref, ab1_ref, xb1_ref,
                gamma_ref, beta_ref,
                wh_ref, bh_ref, wo_ref, bo_ref, o0_ref, o1_ref):
    fs = []
    for agg_ref, ab_ref, xb_ref in ((agg0_ref, ab0_ref, xb0_ref),
                                    (agg1_ref, ab1_ref, xb1_ref)):
        a = agg_ref[...]
        mu = jnp.mean(a, axis=0, keepdims=True)
        var = jnp.mean((a - mu) ** 2, axis=0, keepdims=True)
        nrm = (ab_ref[0] - mu) / jnp.sqrt(var + 1e-5) * gamma_ref[...] \
            + beta_ref[...]
        fs.append(xb_ref[0] + nrm)
    f0, f1 = fs
    for l in range(NPROP):
        wh_t = wh_ref[l, :DIM, :]
        wh_b = wh_ref[l, DIM:, :]
        bh = bh_ref[l]
        wo = wo_ref[l]
        bo = bo_ref[l]
        s = lax.dot_general(f0, f1, (((1,), (1,)), ((), ())),
                            preferred_element_type=jnp.float32)
        m0 = jnp.max(s, axis=1, keepdims=True)
        e0 = jnp.exp(s - m0)
        m1 = jnp.max(s, axis=0, keepdims=True)
        e1 = jnp.exp(s - m1)
        att0 = jnp.dot(e0, f1, preferred_element_type=jnp.float32) \
            / jnp.sum(e0, axis=1, keepdims=True)
        att1 = lax.dot_general(e1, f0, (((0,), (0,)), ((), ())),
                               preferred_element_type=jnp.float32) \
            / jnp.sum(e1, axis=0, keepdims=True).reshape(NP, 1)
        mu0 = f0 - att0
        mu1 = f1 - att1
        h0 = jax.nn.relu(
            jnp.dot(f0, wh_t, preferred_element_type=jnp.float32)
            + jnp.dot(mu0, wh_b, preferred_element_type=jnp.float32) + bh)
        h1 = jax.nn.relu(
            jnp.dot(f1, wh_t, preferred_element_type=jnp.float32)
            + jnp.dot(mu1, wh_b, preferred_element_type=jnp.float32) + bh)
        f0 = f0 + jnp.dot(h0, wo, preferred_element_type=jnp.float32) + bo
        f1 = f1 + jnp.dot(h1, wo, preferred_element_type=jnp.float32) + bo
    o0_ref[0] = f0
    o1_ref[0] = f1


def _cross_call(agg0, x0, agg1, x1, gamma, beta, wh, bh, wo, bo):
    full = pl.BlockSpec((NNODE, DIM), lambda b: (0, 0))
    batch = pl.BlockSpec((1, NP, DIM), lambda b: (b, 0, 0))
    return pl.pallas_call(
        _cross_body,
        grid=(B,),
        in_specs=[
            full, batch, batch,
            full, batch, batch,
            pl.BlockSpec((1, DIM), lambda b: (0, 0)),
            pl.BlockSpec((1, DIM), lambda b: (0, 0)),
            pl.BlockSpec((NPROP, 2 * DIM, HID), lambda b: (0, 0, 0)),
            pl.BlockSpec((NPROP, 1, HID), lambda b: (0, 0, 0)),
            pl.BlockSpec((NPROP, HID, DIM), lambda b: (0, 0, 0)),
            pl.BlockSpec((NPROP, 1, DIM), lambda b: (0, 0, 0)),
        ],
        out_specs=[
            pl.BlockSpec((1, NP, DIM), lambda b: (b, 0, 0)),
            pl.BlockSpec((1, NP, DIM), lambda b: (b, 0, 0)),
        ],
        out_shape=[
            jax.ShapeDtypeStruct((B, NP, DIM), jnp.float32),
            jax.ShapeDtypeStruct((B, NP, DIM), jnp.float32),
        ],
        interpret=_INTERPRET,
    )(agg0, agg0.reshape(B, NP, DIM), x0.reshape(B, NP, DIM),
      agg1, agg1.reshape(B, NP, DIM), x1.reshape(B, NP, DIM),
      gamma, beta, wh, bh, wo, bo)


def kernel(input_xyz, coord_xyz, input_f, coord_f, Wf, bf, Ws, bs,
           gamma, beta, Wh, bh, Wo, bo):
    x0 = input_f.reshape(NNODE, DIM)
    x1 = coord_f.reshape(NNODE, DIM)

    wft, wfb = Wf[:DIM], Wf[DIM:]
    wst, wsb = Ws[:DIM], Ws[DIM:]
    bf2 = bf.reshape(1, DIM)
    bs2 = bs.reshape(1, DIM)
    nt = NNODE // _TN

    aggs = []
    pad = jnp.zeros((B, NP, 5), jnp.float32)
    for xyz, x in ((input_xyz, x0), (coord_xyz, x1)):
        xyzp = jnp.concatenate([xyz, pad], axis=-1)
        xyzt = jnp.swapaxes(xyzp, 1, 2)
        idx = _knn_call(xyzp, xyzt)
        xg = _gather_call(idx.reshape(-1), x)
        agg = _msg_call(
            x.reshape(nt, _TN, DIM),
            xg.reshape(B, K, NP, DIM),
            wft, wfb, wst, wsb, bf2, bs2,
        )
        aggs.append(agg.reshape(NNODE, DIM))

    o0, o1 = _cross_call(aggs[0], x0, aggs[1], x1,
                         gamma.reshape(1, DIM), beta.reshape(1, DIM),
                         Wh, bh.reshape(NPROP, 1, HID),
                         Wo, bo.reshape(NPROP, 1, DIM))
    return o0.reshape(-1, DIM), o1.reshape(-1, DIM)
